# baseline (speedup 1.0000x reference)
import jax
import jax.numpy as jnp
from jax.experimental import pallas as pl
from jax.experimental.pallas import tpu as pltpu


def _linear_kernel(x_ref, w_ref, b_ref, o_ref):
    x = x_ref[...].astype(jnp.float32)
    y = jnp.sum(x * w_ref[...], axis=2)
    o_ref[...] = y.reshape(o_ref.shape) + b_ref[0]


def kernel(x, weight, bias):
    B, K = x.shape
    bias_f32 = bias.astype(jnp.float32).reshape(1)
    w3 = weight.astype(jnp.float32).reshape(1, 1, K)

    x3 = x.reshape(B // 8, 8, K)
    n = B // 8
    tbg = 2048
    grid = (pl.cdiv(n, tbg),)
    rows_out = tbg * 8 // 128

    out = pl.pallas_call(
        _linear_kernel,
        out_shape=jax.ShapeDtypeStruct((B // 128, 128), jnp.float32),
        grid_spec=pltpu.PrefetchScalarGridSpec(
            num_scalar_prefetch=0,
            grid=grid,
            in_specs=[
                pl.BlockSpec((tbg, 8, K), lambda i: (i, 0, 0)),
                pl.BlockSpec((1, 1, K), lambda i: (0, 0, 0)),
                pl.BlockSpec(memory_space=pltpu.MemorySpace.SMEM),
            ],
            out_specs=pl.BlockSpec((rows_out, 128), lambda i: (i, 0)),
        ),
        compiler_params=pltpu.CompilerParams(
            dimension_semantics=("parallel",),
            vmem_limit_bytes=100 * 1024 * 1024,
        ),
    )(x3, w3, bias_f32)
    return out.reshape(B, 1)


# speedup vs baseline: 1.0518x; 1.0518x over previous
import jax
import jax.numpy as jnp
from jax.experimental import pallas as pl
from jax.experimental.pallas import tpu as pltpu


def _linear_kernel(x_ref, w_ref, b_ref, o_ref):
    x = x_ref[...].astype(jnp.float32)
    y = jnp.sum(x * w_ref[...], axis=2)
    o_ref[...] = y.reshape(o_ref.shape) + b_ref[0]


def kernel(x, weight, bias):
    B, K = x.shape
    bias_f32 = bias.astype(jnp.float32).reshape(1)
    w3 = weight.astype(jnp.float32).reshape(1, 1, K)

    x3 = x.reshape(B // 8, 8, K)
    n = B // 8
    tbg = 4096
    grid = (pl.cdiv(n, tbg),)
    rows_out = tbg * 8 // 128

    out = pl.pallas_call(
        _linear_kernel,
        out_shape=jax.ShapeDtypeStruct((B // 128, 128), jnp.float32),
        grid_spec=pltpu.PrefetchScalarGridSpec(
            num_scalar_prefetch=0,
            grid=grid,
            in_specs=[
                pl.BlockSpec((tbg, 8, K), lambda i: (i, 0, 0)),
                pl.BlockSpec((1, 1, K), lambda i: (0, 0, 0)),
                pl.BlockSpec(memory_space=pltpu.MemorySpace.SMEM),
            ],
            out_specs=pl.BlockSpec((rows_out, 128), lambda i: (i, 0)),
        ),
        compiler_params=pltpu.CompilerParams(
            dimension_semantics=("parallel",),
            vmem_limit_bytes=100 * 1024 * 1024,
        ),
    )(x3, w3, bias_f32)
    return out.reshape(B, 1)


# speedup vs baseline: 1.0522x; 1.0004x over previous
import jax
import jax.numpy as jnp
from jax.experimental import pallas as pl
from jax.experimental.pallas import tpu as pltpu


def _linear_kernel(xa_ref, xb_ref, w_ref, b_ref, o_ref):
    half = o_ref.shape[0] // 2
    ya = jnp.sum(xa_ref[...] * w_ref[...], axis=2)
    yb = jnp.sum(xb_ref[...] * w_ref[...], axis=2)
    o_ref[0:half, :] = ya.reshape(half, 128) + b_ref[0]
    o_ref[half:, :] = yb.reshape(half, 128) + b_ref[0]


def kernel(x, weight, bias):
    B, K = x.shape
    bias_f32 = bias.astype(jnp.float32).reshape(1)
    w3 = weight.astype(jnp.float32).reshape(1, 1, K)

    x3 = x.reshape(B // 8, 8, K)
    n = B // 8
    tbg = 4096
    h = tbg // 2
    grid = (pl.cdiv(n, tbg),)
    rows_out = tbg * 8 // 128

    out = pl.pallas_call(
        _linear_kernel,
        out_shape=jax.ShapeDtypeStruct((B // 128, 128), jnp.float32),
        grid_spec=pltpu.PrefetchScalarGridSpec(
            num_scalar_prefetch=0,
            grid=grid,
            in_specs=[
                pl.BlockSpec((h, 8, K), lambda i: (2 * i, 0, 0)),
                pl.BlockSpec((h, 8, K), lambda i: (2 * i + 1, 0, 0)),
                pl.BlockSpec((1, 1, K), lambda i: (0, 0, 0)),
                pl.BlockSpec(memory_space=pltpu.MemorySpace.SMEM),
            ],
            out_specs=pl.BlockSpec((rows_out, 128), lambda i: (i, 0)),
        ),
        compiler_params=pltpu.CompilerParams(
            dimension_semantics=("parallel",),
            vmem_limit_bytes=100 * 1024 * 1024,
        ),
    )(x3, x3, w3, bias_f32)
    return out.reshape(B, 1)
